# baseline (device time: 82628 ns/iter reference)
import jax
import jax.numpy as jnp
from jax import lax
from jax.experimental import pallas as pl
from jax.experimental.pallas import tpu as pltpu


def kernel(x):
    x = x.reshape(x.shape[-2], x.shape[-1])
    m, n = x.shape
    half = m // 2
    q = m // 4
    e = m // 8

    def body(x_ref, out_ref, acc_a, acc_b, comm1, comm2,
             send_sems, recv_sems, cp_sems):
        my_x = lax.axis_index("x")
        my_y = lax.axis_index("y")
        x_nbr = (1 - my_x, my_y)
        y_nbr = (my_x, 1 - my_y)

        barrier_sem = pltpu.get_barrier_semaphore()
        for nbr in (x_nbr, y_nbr):
            pl.semaphore_signal(
                barrier_sem, inc=1,
                device_id=nbr, device_id_type=pl.DeviceIdType.MESH,
            )
        pl.semaphore_wait(barrier_sem, 2)

        a_base = my_y * q
        b_base = half + my_x * q
        ja0 = 1 - my_x
        jb0 = 1 - my_y
        a_mine = a_base + my_x * e
        b_mine = b_base + my_y * e

        def exch(src, dst, sem, nbr):
            return pltpu.make_async_remote_copy(
                src_ref=src, dst_ref=dst,
                send_sem=send_sems.at[sem], recv_sem=recv_sems.at[sem],
                device_id=nbr, device_id_type=pl.DeviceIdType.MESH,
            )

        a1p0 = exch(x_ref.at[pl.ds((1 - my_y) * q + ja0 * e, e), :],
                    comm1.at[0], 0, y_nbr)
        a1p1 = exch(x_ref.at[pl.ds((1 - my_y) * q + (1 - ja0) * e, e), :],
                    comm1.at[1], 1, y_nbr)
        b1p0 = exch(x_ref.at[pl.ds(half + (1 - my_x) * q + jb0 * e, e), :],
                    comm1.at[2], 2, x_nbr)
        b1p1 = exch(x_ref.at[pl.ds(half + (1 - my_x) * q + (1 - jb0) * e, e), :],
                    comm1.at[3], 3, x_nbr)
        a1p0.start()
        a1p1.start()
        b1p0.start()
        b1p1.start()

        cp_a = pltpu.make_async_copy(
            x_ref.at[pl.ds(a_base, q), :], acc_a, cp_sems.at[0])
        cp_b = pltpu.make_async_copy(
            x_ref.at[pl.ds(b_base, q), :], acc_b, cp_sems.at[1])
        cp_a.start()
        cp_b.start()

        a1p0.wait()
        cp_a.wait()
        acc_a[pl.ds(ja0 * e, e), :] = acc_a[pl.ds(ja0 * e, e), :] + comm1[0]
        a2 = exch(acc_a.at[pl.ds(ja0 * e, e), :], comm2.at[0], 4, x_nbr)
        a2.start()

        b1p0.wait()
        cp_b.wait()
        acc_b[pl.ds(jb0 * e, e), :] = acc_b[pl.ds(jb0 * e, e), :] + comm1[2]
        b2 = exch(acc_b.at[pl.ds(jb0 * e, e), :], comm2.at[1], 5, y_nbr)
        b2.start()

        a1p1.wait()
        acc_a[pl.ds((1 - ja0) * e, e), :] = (
            acc_a[pl.ds((1 - ja0) * e, e), :] + comm1[1]
        )
        b1p1.wait()
        acc_b[pl.ds((1 - jb0) * e, e), :] = (
            acc_b[pl.ds((1 - jb0) * e, e), :] + comm1[3]
        )

        a2.wait()
        acc_a[pl.ds(my_x * e, e), :] = acc_a[pl.ds(my_x * e, e), :] + comm2[0]
        a3 = exch(acc_a.at[pl.ds(my_x * e, e), :],
                  out_ref.at[pl.ds(a_mine, e), :], 6, x_nbr)
        a3.start()
        a4a = exch(acc_a.at[pl.ds(my_x * e, e), :],
                   out_ref.at[pl.ds(a_mine, e), :], 8, y_nbr)
        a4a.start()
        cpo_a = pltpu.make_async_copy(
            acc_a.at[pl.ds(my_x * e, e), :],
            out_ref.at[pl.ds(a_mine, e), :], cp_sems.at[2])
        cpo_a.start()

        b2.wait()
        acc_b[pl.ds(my_y * e, e), :] = acc_b[pl.ds(my_y * e, e), :] + comm2[1]
        b3 = exch(acc_b.at[pl.ds(my_y * e, e), :],
                  out_ref.at[pl.ds(b_mine, e), :], 7, y_nbr)
        b3.start()
        b4a = exch(acc_b.at[pl.ds(my_y * e, e), :],
                   out_ref.at[pl.ds(b_mine, e), :], 10, x_nbr)
        b4a.start()
        cpo_b = pltpu.make_async_copy(
            acc_b.at[pl.ds(my_y * e, e), :],
            out_ref.at[pl.ds(b_mine, e), :], cp_sems.at[3])
        cpo_b.start()

        a3.wait()
        a4b = exch(out_ref.at[pl.ds(a_base + (1 - my_x) * e, e), :],
                   out_ref.at[pl.ds(a_base + (1 - my_x) * e, e), :], 9, y_nbr)
        a4b.start()

        b3.wait()
        b4b = exch(out_ref.at[pl.ds(b_base + (1 - my_y) * e, e), :],
                   out_ref.at[pl.ds(b_base + (1 - my_y) * e, e), :], 11, x_nbr)
        b4b.start()

        a4a.wait()
        a4b.wait()
        b4a.wait()
        b4b.wait()
        cpo_a.wait()
        cpo_b.wait()

    return pl.pallas_call(
        body,
        out_shape=jax.ShapeDtypeStruct((m, n), jnp.float32),
        in_specs=[pl.BlockSpec(memory_space=pl.ANY)],
        out_specs=pl.BlockSpec(memory_space=pl.ANY),
        scratch_shapes=[
            pltpu.VMEM((q, n), jnp.float32),
            pltpu.VMEM((q, n), jnp.float32),
            pltpu.VMEM((4, e, n), jnp.float32),
            pltpu.VMEM((2, e, n), jnp.float32),
            pltpu.SemaphoreType.DMA((12,)),
            pltpu.SemaphoreType.DMA((12,)),
            pltpu.SemaphoreType.DMA((4,)),
        ],
        compiler_params=pltpu.CompilerParams(collective_id=0),
    )(x)


# device time: 79438 ns/iter; 1.0402x vs baseline; 1.0402x over previous
import jax
import jax.numpy as jnp
from jax import lax
from jax.experimental import pallas as pl
from jax.experimental.pallas import tpu as pltpu


def kernel(x):
    x = x.reshape(x.shape[-2], x.shape[-1])
    m, n = x.shape
    half = m // 2
    q = m // 4
    e = m // 8
    h = m // 16

    def body(x_ref, out_ref, acc_a, acc_b, comm1, comm2,
             send_sems, recv_sems, cp_sems):
        my_x = lax.axis_index("x")
        my_y = lax.axis_index("y")
        x_nbr = (1 - my_x, my_y)
        y_nbr = (my_x, 1 - my_y)

        barrier_sem = pltpu.get_barrier_semaphore()
        for nbr in (x_nbr, y_nbr):
            pl.semaphore_signal(
                barrier_sem, inc=1,
                device_id=nbr, device_id_type=pl.DeviceIdType.MESH,
            )
        pl.semaphore_wait(barrier_sem, 2)

        a_base = my_y * q
        b_base = half + my_x * q
        ja0 = 1 - my_x
        jb0 = 1 - my_y
        a_mine = a_base + my_x * e
        b_mine = b_base + my_y * e

        def exch(src, dst, sem, nbr):
            return pltpu.make_async_remote_copy(
                src_ref=src, dst_ref=dst,
                send_sem=send_sems.at[sem], recv_sem=recv_sems.at[sem],
                device_id=nbr, device_id_type=pl.DeviceIdType.MESH,
            )

        def rs1_off(axis0_chunk, j):
            c = axis0_chunk if j < 2 else 1 - axis0_chunk
            return c * e + (j % 2) * h

        a1 = []
        b1 = []
        for j in range(4):
            a1.append(exch(
                x_ref.at[pl.ds((1 - my_y) * q + rs1_off(ja0, j), h), :],
                comm1.at[j], j, y_nbr))
            b1.append(exch(
                x_ref.at[pl.ds(half + (1 - my_x) * q + rs1_off(jb0, j), h), :],
                comm1.at[4 + j], 4 + j, x_nbr))
        for r in a1:
            r.start()
        for r in b1:
            r.start()

        cp_a = pltpu.make_async_copy(
            x_ref.at[pl.ds(a_base, q), :], acc_a, cp_sems.at[0])
        cp_b = pltpu.make_async_copy(
            x_ref.at[pl.ds(b_base, q), :], acc_b, cp_sems.at[1])
        cp_a.start()
        cp_b.start()

        a2 = []
        b2 = []
        cp_a.wait()
        cp_b.wait()
        for j in range(4):
            oa = rs1_off(ja0, j)
            a1[j].wait()
            acc_a[pl.ds(oa, h), :] = acc_a[pl.ds(oa, h), :] + comm1[j]
            if j < 2:
                a2.append(exch(acc_a.at[pl.ds(oa, h), :],
                               comm2.at[j], 8 + j, x_nbr))
                a2[j].start()
            ob = rs1_off(jb0, j)
            b1[j].wait()
            acc_b[pl.ds(ob, h), :] = acc_b[pl.ds(ob, h), :] + comm1[4 + j]
            if j < 2:
                b2.append(exch(acc_b.at[pl.ds(ob, h), :],
                               comm2.at[2 + j], 10 + j, y_nbr))
                b2[j].start()

        a3 = []
        b3 = []
        a4 = []
        b4 = []
        cpo = []
        for k in range(2):
            ra = my_x * e + k * h
            a2[k].wait()
            acc_a[pl.ds(ra, h), :] = acc_a[pl.ds(ra, h), :] + comm2[k]
            a3.append(exch(acc_a.at[pl.ds(ra, h), :],
                           out_ref.at[pl.ds(a_mine + k * h, h), :],
                           12 + k, x_nbr))
            a3[k].start()
            a4.append(exch(acc_a.at[pl.ds(ra, h), :],
                           out_ref.at[pl.ds(a_mine + k * h, h), :],
                           16 + k, y_nbr))
            a4[-1].start()
            cpo.append(pltpu.make_async_copy(
                acc_a.at[pl.ds(ra, h), :],
                out_ref.at[pl.ds(a_mine + k * h, h), :],
                cp_sems.at[2 + k]))
            cpo[-1].start()

            rb = my_y * e + k * h
            b2[k].wait()
            acc_b[pl.ds(rb, h), :] = acc_b[pl.ds(rb, h), :] + comm2[2 + k]
            b3.append(exch(acc_b.at[pl.ds(rb, h), :],
                           out_ref.at[pl.ds(b_mine + k * h, h), :],
                           14 + k, y_nbr))
            b3[k].start()
            b4.append(exch(acc_b.at[pl.ds(rb, h), :],
                           out_ref.at[pl.ds(b_mine + k * h, h), :],
                           20 + k, x_nbr))
            b4[-1].start()
            cpo.append(pltpu.make_async_copy(
                acc_b.at[pl.ds(rb, h), :],
                out_ref.at[pl.ds(b_mine + k * h, h), :],
                cp_sems.at[4 + k]))
            cpo[-1].start()

        for k in range(2):
            a3[k].wait()
            a4.append(exch(
                out_ref.at[pl.ds(a_base + (1 - my_x) * e + k * h, h), :],
                out_ref.at[pl.ds(a_base + (1 - my_x) * e + k * h, h), :],
                18 + k, y_nbr))
            a4[-1].start()
            b3[k].wait()
            b4.append(exch(
                out_ref.at[pl.ds(b_base + (1 - my_y) * e + k * h, h), :],
                out_ref.at[pl.ds(b_base + (1 - my_y) * e + k * h, h), :],
                22 + k, x_nbr))
            b4[-1].start()

        for r in a4:
            r.wait()
        for r in b4:
            r.wait()
        for c in cpo:
            c.wait()

    return pl.pallas_call(
        body,
        out_shape=jax.ShapeDtypeStruct((m, n), jnp.float32),
        in_specs=[pl.BlockSpec(memory_space=pl.ANY)],
        out_specs=pl.BlockSpec(memory_space=pl.ANY),
        scratch_shapes=[
            pltpu.VMEM((q, n), jnp.float32),
            pltpu.VMEM((q, n), jnp.float32),
            pltpu.VMEM((8, h, n), jnp.float32),
            pltpu.VMEM((4, h, n), jnp.float32),
            pltpu.SemaphoreType.DMA((24,)),
            pltpu.SemaphoreType.DMA((24,)),
            pltpu.SemaphoreType.DMA((6,)),
        ],
        compiler_params=pltpu.CompilerParams(collective_id=0),
    )(x)


# device time: 78724 ns/iter; 1.0496x vs baseline; 1.0091x over previous
import jax
import jax.numpy as jnp
from jax import lax
from jax.experimental import pallas as pl
from jax.experimental.pallas import tpu as pltpu

G = 4


def kernel(x):
    x = x.reshape(x.shape[-2], x.shape[-1])
    m, n = x.shape
    half = m // 2
    q = m // 4
    e = m // 8
    h = e // G

    def body(x_ref, out_ref, acc_a, acc_b, comm1, comm2,
             send_sems, recv_sems, cp_sems):
        my_x = lax.axis_index("x")
        my_y = lax.axis_index("y")
        x_nbr = (1 - my_x, my_y)
        y_nbr = (my_x, 1 - my_y)

        barrier_sem = pltpu.get_barrier_semaphore()
        for nbr in (x_nbr, y_nbr):
            pl.semaphore_signal(
                barrier_sem, inc=1,
                device_id=nbr, device_id_type=pl.DeviceIdType.MESH,
            )
        pl.semaphore_wait(barrier_sem, 2)

        a_base = my_y * q
        b_base = half + my_x * q
        ja0 = 1 - my_x
        jb0 = 1 - my_y
        a_mine = a_base + my_x * e
        b_mine = b_base + my_y * e

        def exch(src, dst, sem, nbr):
            return pltpu.make_async_remote_copy(
                src_ref=src, dst_ref=dst,
                send_sem=send_sems.at[sem], recv_sem=recv_sems.at[sem],
                device_id=nbr, device_id_type=pl.DeviceIdType.MESH,
            )

        def rs1_off(first_chunk, j):
            c = first_chunk if j < G else 1 - first_chunk
            return c * e + (j % G) * h

        a1 = []
        b1 = []
        for j in range(2 * G):
            a1.append(exch(
                x_ref.at[pl.ds((1 - my_y) * q + rs1_off(ja0, j), h), :],
                comm1.at[j], j, y_nbr))
            b1.append(exch(
                x_ref.at[pl.ds(half + (1 - my_x) * q + rs1_off(jb0, j), h), :],
                comm1.at[2 * G + j], 2 * G + j, x_nbr))
        for r in a1:
            r.start()
        for r in b1:
            r.start()

        cp_a = pltpu.make_async_copy(
            x_ref.at[pl.ds(a_base, q), :], acc_a, cp_sems.at[0])
        cp_b = pltpu.make_async_copy(
            x_ref.at[pl.ds(b_base, q), :], acc_b, cp_sems.at[1])
        cp_a.start()
        cp_b.start()

        a2 = []
        b2 = []
        cp_a.wait()
        cp_b.wait()
        for j in range(2 * G):
            oa = rs1_off(ja0, j)
            a1[j].wait()
            acc_a[pl.ds(oa, h), :] = acc_a[pl.ds(oa, h), :] + comm1[j]
            if j < G:
                a2.append(exch(acc_a.at[pl.ds(oa, h), :],
                               comm2.at[j], 4 * G + j, x_nbr))
                a2[j].start()
            ob = rs1_off(jb0, j)
            b1[j].wait()
            acc_b[pl.ds(ob, h), :] = (
                acc_b[pl.ds(ob, h), :] + comm1[2 * G + j]
            )
            if j < G:
                b2.append(exch(acc_b.at[pl.ds(ob, h), :],
                               comm2.at[G + j], 5 * G + j, y_nbr))
                b2[j].start()

        a3 = []
        b3 = []
        a4 = []
        b4 = []
        cpo = []
        for k in range(G):
            ra = my_x * e + k * h
            a2[k].wait()
            acc_a[pl.ds(ra, h), :] = acc_a[pl.ds(ra, h), :] + comm2[k]
            a3.append(exch(acc_a.at[pl.ds(ra, h), :],
                           out_ref.at[pl.ds(a_mine + k * h, h), :],
                           6 * G + k, x_nbr))
            a3[k].start()
            a4.append(exch(acc_a.at[pl.ds(ra, h), :],
                           out_ref.at[pl.ds(a_mine + k * h, h), :],
                           8 * G + k, y_nbr))
            a4[-1].start()
            cpo.append(pltpu.make_async_copy(
                acc_a.at[pl.ds(ra, h), :],
                out_ref.at[pl.ds(a_mine + k * h, h), :],
                cp_sems.at[2 + k]))
            cpo[-1].start()

            rb = my_y * e + k * h
            b2[k].wait()
            acc_b[pl.ds(rb, h), :] = acc_b[pl.ds(rb, h), :] + comm2[G + k]
            b3.append(exch(acc_b.at[pl.ds(rb, h), :],
                           out_ref.at[pl.ds(b_mine + k * h, h), :],
                           7 * G + k, y_nbr))
            b3[k].start()
            b4.append(exch(acc_b.at[pl.ds(rb, h), :],
                           out_ref.at[pl.ds(b_mine + k * h, h), :],
                           10 * G + k, x_nbr))
            b4[-1].start()
            cpo.append(pltpu.make_async_copy(
                acc_b.at[pl.ds(rb, h), :],
                out_ref.at[pl.ds(b_mine + k * h, h), :],
                cp_sems.at[2 + G + k]))
            cpo[-1].start()

        for k in range(G):
            a3[k].wait()
            a4.append(exch(
                out_ref.at[pl.ds(a_base + (1 - my_x) * e + k * h, h), :],
                out_ref.at[pl.ds(a_base + (1 - my_x) * e + k * h, h), :],
                9 * G + k, y_nbr))
            a4[-1].start()
            b3[k].wait()
            b4.append(exch(
                out_ref.at[pl.ds(b_base + (1 - my_y) * e + k * h, h), :],
                out_ref.at[pl.ds(b_base + (1 - my_y) * e + k * h, h), :],
                11 * G + k, x_nbr))
            b4[-1].start()

        for r in a4:
            r.wait()
        for r in b4:
            r.wait()
        for c in cpo:
            c.wait()

    return pl.pallas_call(
        body,
        out_shape=jax.ShapeDtypeStruct((m, n), jnp.float32),
        in_specs=[pl.BlockSpec(memory_space=pl.ANY)],
        out_specs=pl.BlockSpec(memory_space=pl.ANY),
        scratch_shapes=[
            pltpu.VMEM((q, n), jnp.float32),
            pltpu.VMEM((q, n), jnp.float32),
            pltpu.VMEM((4 * G, h, n), jnp.float32),
            pltpu.VMEM((2 * G, h, n), jnp.float32),
            pltpu.SemaphoreType.DMA((12 * G,)),
            pltpu.SemaphoreType.DMA((12 * G,)),
            pltpu.SemaphoreType.DMA((2 + 2 * G,)),
        ],
        compiler_params=pltpu.CompilerParams(collective_id=0),
    )(x)
